# Initial kernel scaffold; baseline (speedup 1.0000x reference)
import jax
import jax.numpy as jnp
from jax.experimental import pallas as pl


def kernel(x, edge_index, W1, W2, w, ln1_g, ln1_b, ln2_g, ln2_b, Wo1, Wo2):
    raise NotImplementedError("write your pallas kernel here")


# speedup vs baseline: 4.3559x; 4.3559x over previous
import functools

import jax
import jax.numpy as jnp
from jax import lax
from jax.experimental import pallas as pl
from jax.experimental.pallas import tpu as pltpu
from jax.experimental.pallas import tpu_sc as plsc

N = 10000
E = 320000
D = 128
DP = 144
NC = 2
NS = 16
NW = NC * NS
CH = 128
EPT = 10112
E_PAD = EPT * NW
N_ACC = 10112
DUMMY = 10048
ROWS_PER_TILE = N_ACC // NS


def _mlp_body(x_ref, w1_ref, w2_ref, out_ref):
    h = jnp.maximum(jnp.dot(x_ref[...], w1_ref[...],
                            preferred_element_type=jnp.float32), 0.0)
    h = jnp.dot(h, w2_ref[...], preferred_element_type=jnp.float32)
    b = h.shape[0]
    ones = jnp.ones((b, 1), jnp.float32)
    zeros = jnp.zeros((b, DP - D - 1), jnp.float32)
    out_ref[...] = jnp.concatenate([h, ones, zeros], axis=1)


def _node_mlp(x, W1, W2):
    BM = 2000
    return pl.pallas_call(
        _mlp_body,
        grid=(N // BM,),
        in_specs=[pl.BlockSpec((BM, D), lambda i: (i, 0)),
                  pl.BlockSpec((D, D), lambda i: (0, 0)),
                  pl.BlockSpec((D, D), lambda i: (0, 0))],
        out_specs=pl.BlockSpec((BM, DP), lambda i: (i, 0)),
        out_shape=jax.ShapeDtypeStruct((N, DP), jnp.float32),
    )(x, W1, W2)


def _seg_body(g_hbm, row_hbm, col_hbm, zero_hbm, out_hbm,
              rows_v, ridx_v, cidx_v, acc_sh, sem):
    cid = lax.axis_index("c")
    sid = lax.axis_index("s")
    wid = sid * NC + cid

    zbase = sid * ROWS_PER_TILE
    pltpu.sync_copy(zero_hbm.at[pl.ds(zbase, ROWS_PER_TILE)],
                    acc_sh.at[pl.ds(zbase, ROWS_PER_TILE)])
    plsc.subcore_barrier()

    ebase = wid * EPT

    @pl.loop(0, EPT // CH)
    def _chunk(c):
        o = pl.multiple_of(ebase + c * CH, 8)
        pltpu.sync_copy(row_hbm.at[pl.ds(o, CH)], ridx_v)
        pltpu.sync_copy(col_hbm.at[pl.ds(o, CH)], cidx_v)
        pltpu.async_copy(g_hbm.at[ridx_v], rows_v, sem).wait()
        pltpu.sync_copy(rows_v, acc_sh.at[cidx_v], add=True)

    plsc.subcore_barrier()
    pltpu.sync_copy(acc_sh.at[pl.ds(zbase, ROWS_PER_TILE)],
                    out_hbm.at[cid, pl.ds(zbase, ROWS_PER_TILE)])


@functools.cache
def _make_seg_sum():
    return pl.kernel(
        _seg_body,
        out_type=jax.ShapeDtypeStruct((NC, N_ACC, DP), jnp.float32),
        mesh=plsc.VectorSubcoreMesh(core_axis_name="c", subcore_axis_name="s",
                                    num_cores=NC, num_subcores=NS),
        scratch_types=[
            pltpu.VMEM((CH, DP), jnp.float32),
            pltpu.VMEM((CH,), jnp.int32),
            pltpu.VMEM((CH,), jnp.int32),
            pltpu.VMEM_SHARED((N_ACC, DP), jnp.float32),
            pltpu.SemaphoreType.DMA,
        ],
        compiler_params=pltpu.CompilerParams(use_tc_tiling_on_sc=False),
    )


def _post_body(a0_ref, a1_ref, x_ref, w_ref, g1_ref, b1_ref, g2_ref, b2_ref,
               wo1a_ref, wo1b_ref, wo2_ref, out_ref):
    a = a0_ref[...] + a1_ref[...]
    sums = a[:, :D]
    cnt = a[:, D:D + 1]
    agg = sums / jnp.maximum(cnt, 1.0)
    m1 = jnp.mean(agg, axis=-1, keepdims=True)
    v1 = jnp.mean((agg - m1) ** 2, axis=-1, keepdims=True)
    agg_n = (agg - m1) * lax.rsqrt(v1 + 1e-5) * g1_ref[...] + b1_ref[...]
    x = x_ref[...]
    y = x + (x - agg_n) * w_ref[...]
    m2 = jnp.mean(y, axis=-1, keepdims=True)
    v2 = jnp.mean((y - m2) ** 2, axis=-1, keepdims=True)
    fx = (y - m2) * lax.rsqrt(v2 + 1e-5) * g2_ref[...] + b2_ref[...]
    h = jnp.maximum(jnp.dot(fx, wo1a_ref[...], preferred_element_type=jnp.float32)
                    + jnp.dot(agg_n, wo1b_ref[...], preferred_element_type=jnp.float32),
                    0.0)
    out_ref[...] = jnp.dot(h, wo2_ref[...], preferred_element_type=jnp.float32)


def _post(a0, a1, x, w, ln1_g, ln1_b, ln2_g, ln2_b, Wo1a, Wo1b, Wo2):
    BM = 2000
    vec = lambda: pl.BlockSpec((1, D), lambda i: (0, 0))
    mat = lambda: pl.BlockSpec((D, D), lambda i: (0, 0))
    return pl.pallas_call(
        _post_body,
        grid=(N // BM,),
        in_specs=[pl.BlockSpec((BM, DP), lambda i: (i, 0)),
                  pl.BlockSpec((BM, DP), lambda i: (i, 0)),
                  pl.BlockSpec((BM, D), lambda i: (i, 0)),
                  vec(), vec(), vec(), vec(), vec(),
                  mat(), mat(), mat()],
        out_specs=pl.BlockSpec((BM, D), lambda i: (i, 0)),
        out_shape=jax.ShapeDtypeStruct((N, D), jnp.float32),
    )(a0, a1, x, w, ln1_g, ln1_b, ln2_g, ln2_b, Wo1a, Wo1b, Wo2)


def kernel(x, edge_index, W1, W2, w, ln1_g, ln1_b, ln2_g, ln2_b, Wo1, Wo2):
    row = edge_index[0].astype(jnp.int32)
    col = edge_index[1].astype(jnp.int32)
    pad = E_PAD - E
    row_p = jnp.concatenate([row, jnp.zeros((pad,), jnp.int32)])
    col_p = jnp.concatenate([col, jnp.full((pad,), DUMMY, jnp.int32)])
    zero = jnp.zeros((N_ACC, DP), jnp.float32)

    g = _node_mlp(x, W1, W2)
    partials = _make_seg_sum()(g, row_p, col_p, zero)

    a0 = partials[0, :N]
    a1 = partials[1, :N]
    return _post(a0, a1, x,
                 w.reshape(1, D),
                 ln1_g.reshape(1, D), ln1_b.reshape(1, D),
                 ln2_g.reshape(1, D), ln2_b.reshape(1, D),
                 Wo1[:D], Wo1[D:], Wo2)
